# baseline (speedup 1.0000x reference)
import functools

import jax
import jax.numpy as jnp
from jax import lax
from jax.experimental import pallas as pl
from jax.experimental.pallas import tpu as pltpu
from jax.experimental.pallas import tpu_sc as plsc

N = 10000
E = 320000
D = 128
G = 64

NC = 2
TPC = 16
NB = 80
NB0 = 40
NB1 = 120
EP = NC * TPC * NB * 128
NP_ = 10240
PAD = NP_ - 1
NPADE = EP - E
RPT = NP_ // TPC
DH = 64

_f32 = jnp.float32
_i32 = jnp.int32

_MESH = plsc.VectorSubcoreMesh(core_axis_name="c", subcore_axis_name="s")
_SC_PARAMS = pltpu.CompilerParams(needs_layout_passes=False,
                                  use_tc_tiling_on_sc=False)


@functools.partial(
    pl.kernel,
    out_type=jax.ShapeDtypeStruct((NC, NP_), _f32),
    mesh=_MESH,
    scratch_types=[
        pltpu.VMEM((NP_,), _f32),
        pltpu.VMEM((NB, 128), _i32),
        pltpu.VMEM((RPT,), _f32),
        pltpu.VMEM((RPT,), _f32),
        pltpu.VMEM_SHARED((TPC, NP_), _f32),
    ],
    compiler_params=_SC_PARAMS,
)
def _hist_kernel(dst2d, deg_out, hist_v, blk_v, col_v, tmp_v, shared):
    c = lax.axis_index("c")
    s = lax.axis_index("s")
    wid = c * TPC + s
    z16 = jnp.zeros((16,), _f32)
    ones = jnp.ones((16,), _f32)

    def zero_hist(i, _):
        hist_v[pl.ds(i * 16, 16)] = z16
        return 0
    lax.fori_loop(0, NP_ // 16, zero_hist, 0)

    pltpu.sync_copy(dst2d.at[pl.ds(wid * NB, NB)], blk_v)

    def row(r, _):
        def chunk(k, _):
            idx = blk_v[r, pl.ds(k * 16, 16)]
            plsc.addupdate_scatter(hist_v, [idx], ones)
            return 0
        lax.fori_loop(0, 128 // 16, chunk, 0)
        return 0
    lax.fori_loop(0, NB, row, 0)

    pltpu.sync_copy(hist_v, shared.at[s])
    plsc.subcore_barrier()

    def zero_col(i, _):
        col_v[pl.ds(i * 16, 16)] = z16
        return 0
    lax.fori_loop(0, RPT // 16, zero_col, 0)

    base = s * RPT

    def acc_t(t, _):
        pltpu.sync_copy(shared.at[t, pl.ds(base, RPT)], tmp_v)

        def add(i, _):
            sl = pl.ds(i * 16, 16)
            col_v[sl] = col_v[sl] + tmp_v[sl]
            return 0
        lax.fori_loop(0, RPT // 16, add, 0)
        return 0
    lax.fori_loop(0, TPC, acc_t, 0)

    pltpu.sync_copy(col_v, deg_out.at[c, pl.ds(base, RPT)])


@functools.partial(
    pl.kernel,
    out_type=jax.ShapeDtypeStruct((NC, NP_, DH), _f32),
    mesh=_MESH,
    scratch_types=[
        pltpu.VMEM((NB1, 128), _i32),
        pltpu.VMEM((NB1, 128), _i32),
        pltpu.VMEM((128, DH), _f32),
        pltpu.VMEM((128, DH), _f32),
        pltpu.VMEM_SHARED((NP_, DH), _f32),
        pltpu.SemaphoreType.DMA,
        pltpu.SemaphoreType.DMA,
    ],
    compiler_params=_SC_PARAMS,
)
def _scatter_kernel(src2d, dst2d, table, accp, src_v, dst_v, rowa, rowb,
                    shared, gsa, gsb):
    c = lax.axis_index("c")
    s = lax.axis_index("s")
    nb = jnp.where(c == 0, NB0, NB1)
    row0 = c * TPC * NB0 + s * nb
    z16 = jnp.zeros((16,), _f32)
    base = s * RPT

    def zr(i, _):
        def zc(k, _):
            rowa[i, pl.ds(k * 16, 16)] = z16
            return 0
        lax.fori_loop(0, DH // 16, zc, 0)
        return 0
    lax.fori_loop(0, 128, zr, 0)

    def zs(q, _):
        pltpu.sync_copy(rowa, shared.at[pl.ds(base + q * 128, 128)])
        return 0
    lax.fori_loop(0, RPT // 128, zs, 0)
    plsc.subcore_barrier()

    pltpu.sync_copy(src2d.at[pl.ds(row0, NB1)], src_v)
    pltpu.sync_copy(dst2d.at[pl.ds(row0, NB1)], dst_v)

    def pro(t, _):
        pltpu.async_copy(table.at[src_v.at[t]], rowa, gsa)
        return 0
    lax.fori_loop(0, 1, pro, 0)

    def eb(jj, _):
        j = jj * 2
        pltpu.make_async_copy(table.at[src_v.at[j]], rowa, gsa).wait()
        pltpu.async_copy(table.at[src_v.at[j + 1]], rowb, gsb)
        pltpu.sync_copy(rowa, shared.at[dst_v.at[j]], add=True)
        pltpu.make_async_copy(table.at[src_v.at[j + 1]], rowb, gsb).wait()
        pltpu.async_copy(table.at[src_v.at[j + 2]], rowa, gsa)
        pltpu.sync_copy(rowb, shared.at[dst_v.at[j + 1]], add=True)
        return 0
    lax.fori_loop(0, nb // 2 - 1, eb, 0)

    def epi(t, _):
        j = nb - 2 + t
        pltpu.make_async_copy(table.at[src_v.at[j]], rowa, gsa).wait()
        pltpu.async_copy(table.at[src_v.at[j + 1]], rowb, gsb)
        pltpu.sync_copy(rowa, shared.at[dst_v.at[j]], add=True)
        pltpu.make_async_copy(table.at[src_v.at[j + 1]], rowb, gsb).wait()
        pltpu.sync_copy(rowb, shared.at[dst_v.at[j + 1]], add=True)
        return 0
    lax.fori_loop(0, 1, epi, 0)
    plsc.subcore_barrier()

    def co(q, _):
        pltpu.sync_copy(shared.at[pl.ds(base + q * 128, 128)], rowa)
        pltpu.sync_copy(rowa, accp.at[c, pl.ds(base + q * 128, 128)])
        return 0
    lax.fori_loop(0, RPT // 128, co, 0)


def _lnt(x, s, b, eps=1e-5):
    m = jnp.mean(x, axis=-1, keepdims=True)
    v = jnp.mean((x - m) ** 2, axis=-1, keepdims=True)
    return (x - m) * lax.rsqrt(v + eps) * s + b


def _tc_b(xp, w1, degp, hl_o, hr_o, dinv_o):
    deg = degp[:, 0:1] + degp[:, 1:2]
    ii = lax.broadcasted_iota(_i32, (NP_, 1), 0)
    deg = deg + jnp.where(ii < N, 1.0, 0.0) - jnp.where(ii == PAD, float(NPADE), 0.0)
    dinv = jnp.where(deg > 0, lax.rsqrt(deg), 0.0)
    h1 = dinv * jnp.dot(xp[...], w1[...], preferred_element_type=_f32)
    hl_o[...] = h1[:, :DH]
    hr_o[...] = h1[:, DH:]
    dinv_o[...] = dinv


_TC_PARAMS = pltpu.CompilerParams(vmem_limit_bytes=100 * 1024 * 1024)

_B_CALL = pl.pallas_call(
    _tc_b,
    out_shape=[jax.ShapeDtypeStruct((NP_, DH), _f32),
               jax.ShapeDtypeStruct((NP_, DH), _f32),
               jax.ShapeDtypeStruct((NP_, 1), _f32)],
    compiler_params=_TC_PARAMS,
)


def _tc_c(accl, accr, hl, hr, dinv, w2, b1, s1, bb1, h2l_o, h2r_o):
    a = jnp.concatenate([accl[0] + accl[1] + hl[...],
                         accr[0] + accr[1] + hr[...]], axis=1)
    out1 = dinv[...] * a + b1[...]
    h2 = _lnt(jnp.tanh(out1), s1[...], bb1[...])
    h2p = dinv[...] * jnp.dot(h2, w2[...], preferred_element_type=_f32)
    h2l_o[...] = h2p[:, :DH]
    h2r_o[...] = h2p[:, DH:]


_C_CALL = pl.pallas_call(
    _tc_c,
    out_shape=[jax.ShapeDtypeStruct((NP_, DH), _f32),
               jax.ShapeDtypeStruct((NP_, DH), _f32)],
    compiler_params=_TC_PARAMS,
)


def _tc_d(accl, accr, h2l, h2r, dinv, bcol, brow,
          b2, s2, bb2, sp, bp,
          gw1, gb1, gw2, gb2, gw3, gb3, gw4r, gb4,
          mw1, mb1, ms1, mbb1, mw2, mb2, ms2, mbb2, mw3, mb3,
          out_ref):
    a = jnp.concatenate([accl[0] + accl[1] + h2l[...],
                         accr[0] + accr[1] + h2r[...]], axis=1)
    out2 = dinv[...] * a + b2[...]
    h3 = _lnt(jnp.tanh(out2), s2[...], bb2[...])
    hp = _lnt(h3, sp[...], bp[...])
    g = jnp.tanh(jnp.dot(hp, gw1[...], preferred_element_type=_f32) + gb1[...])
    g = jnp.tanh(jnp.dot(g, gw2[...], preferred_element_type=_f32) + gb2[...])
    g = jnp.tanh(jnp.dot(g, gw3[...], preferred_element_type=_f32) + gb3[...])
    g = jnp.sum(g * gw4r[...], axis=1, keepdims=True) + gb4[0, 0]
    iota_row = lax.broadcasted_iota(_i32, (1, G), 1)
    maskT = (bcol[...] == iota_row).astype(_f32)
    gmax = jnp.max(jnp.where(maskT > 0, g, -jnp.inf), axis=0, keepdims=True)
    gmax = jnp.where(gmax > -1e30, gmax, 0.0)
    gmax_n = jnp.sum(maskT * gmax, axis=1, keepdims=True)
    e = jnp.exp(g - gmax_n)
    esum = jnp.sum(maskT * e, axis=0, keepdims=True)
    esum_n = jnp.sum(maskT * esum, axis=1, keepdims=True)
    alpha = e / (esum_n + 1e-16)
    iota_col = lax.broadcasted_iota(_i32, (G, 1), 0)
    mask2 = (brow[...] == iota_col).astype(_f32)
    pooled = jnp.dot(mask2, alpha * hp, preferred_element_type=_f32)
    m = jnp.dot(pooled, mw1[...], preferred_element_type=_f32) + mb1[...]
    m = jnp.tanh(_lnt(m, ms1[...], mbb1[...]))
    m = jnp.dot(m, mw2[...], preferred_element_type=_f32) + mb2[...]
    m = jnp.tanh(_lnt(m, ms2[...], mbb2[...]))
    out_ref[...] = jnp.dot(m, mw3[...], preferred_element_type=_f32) + mb3[...]


_D_CALL = pl.pallas_call(
    _tc_d,
    out_shape=jax.ShapeDtypeStruct((G, G), _f32),
    compiler_params=_TC_PARAMS,
)


def _row(v):
    return v.reshape(1, -1)


def kernel(x, edge_index, batch, params):
    p = params
    src = edge_index[0].astype(_i32)
    dst = edge_index[1].astype(_i32)
    padi = jnp.full((NPADE,), PAD, _i32)
    src2d = jnp.concatenate([src, padi]).reshape(NC * TPC * NB, 128)
    dst2d = jnp.concatenate([dst, padi]).reshape(NC * TPC * NB, 128)
    xp = jnp.zeros((NP_, D), _f32).at[:N].set(x)
    bpad = jnp.concatenate([batch.astype(_i32), jnp.full((NP_ - N,), -1, _i32)])
    bcol = bpad.reshape(NP_, 1)
    brow = bpad.reshape(1, NP_)

    degp = _hist_kernel(dst2d)
    h1l, h1r, dinv = _B_CALL(xp, p['W1'], degp.T)
    a1l = _scatter_kernel(src2d, dst2d, h1l)
    a1r = _scatter_kernel(src2d, dst2d, h1r)
    h2l, h2r = _C_CALL(a1l, a1r, h1l, h1r, dinv, p['W2'],
                       _row(p['b1']), _row(p['ln1_s']), _row(p['ln1_b']))
    a2l = _scatter_kernel(src2d, dst2d, h2l)
    a2r = _scatter_kernel(src2d, dst2d, h2r)
    out = _D_CALL(a2l, a2r, h2l, h2r, dinv, bcol, brow,
                  _row(p['b2']), _row(p['ln2_s']), _row(p['ln2_b']),
                  _row(p['pn_s']), _row(p['pn_b']),
                  p['Gw1'], _row(p['Gb1']), p['Gw2'], _row(p['Gb2']),
                  p['Gw3'], _row(p['Gb3']), _row(p['Gw4'][:, 0]), p['Gb4'].reshape(1, 1),
                  p['Mw1'], _row(p['Mb1']), _row(p['Mln1_s']), _row(p['Mln1_b']),
                  p['Mw2'], _row(p['Mb2']), _row(p['Mln2_s']), _row(p['Mln2_b']),
                  p['Mw3'], _row(p['Mb3']))
    return out


# speedup vs baseline: 1.3298x; 1.3298x over previous
import functools

import jax
import jax.numpy as jnp
from jax import lax
from jax.experimental import pallas as pl
from jax.experimental.pallas import tpu as pltpu
from jax.experimental.pallas import tpu_sc as plsc

N = 10000
E = 320000
D = 128
G = 64

NC = 2
TPC = 16
NB = 80
NB0 = 120
NB1 = 40
NBMAX = max(NB0, NB1)
EROWS = NC * TPC * NB
XROWS = NBMAX - min(NB0, NB1)
EP = NC * TPC * NB * 128
NP_ = 10240
PAD = NP_ - 1
NPADE = EP - E
RPT = NP_ // TPC
DH = 64

_f32 = jnp.float32
_i32 = jnp.int32

_MESH = plsc.VectorSubcoreMesh(core_axis_name="c", subcore_axis_name="s")
_SC_PARAMS = pltpu.CompilerParams(needs_layout_passes=False,
                                  use_tc_tiling_on_sc=False)


@functools.partial(
    pl.kernel,
    out_type=jax.ShapeDtypeStruct((NC, NP_), _f32),
    mesh=_MESH,
    scratch_types=[
        pltpu.VMEM((NP_,), _f32),
        pltpu.VMEM((NB, 128), _i32),
        pltpu.VMEM((RPT,), _f32),
        pltpu.VMEM((RPT,), _f32),
        pltpu.VMEM_SHARED((TPC, NP_), _f32),
    ],
    compiler_params=_SC_PARAMS,
)
def _hist_kernel(dst2d, deg_out, hist_v, blk_v, col_v, tmp_v, shared):
    c = lax.axis_index("c")
    s = lax.axis_index("s")
    wid = c * TPC + s
    z16 = jnp.zeros((16,), _f32)
    ones = jnp.ones((16,), _f32)

    def zero_hist(i, _):
        hist_v[pl.ds(i * 16, 16)] = z16
        return 0
    lax.fori_loop(0, NP_ // 16, zero_hist, 0)

    pltpu.sync_copy(dst2d.at[pl.ds(wid * NB, NB)], blk_v)

    def row(r, _):
        def chunk(k, _):
            idx = blk_v[r, pl.ds(k * 16, 16)]
            plsc.addupdate_scatter(hist_v, [idx], ones)
            return 0
        lax.fori_loop(0, 128 // 16, chunk, 0)
        return 0
    lax.fori_loop(0, NB, row, 0)

    pltpu.sync_copy(hist_v, shared.at[s])
    plsc.subcore_barrier()

    def zero_col(i, _):
        col_v[pl.ds(i * 16, 16)] = z16
        return 0
    lax.fori_loop(0, RPT // 16, zero_col, 0)

    base = s * RPT

    def acc_t(t, _):
        pltpu.sync_copy(shared.at[t, pl.ds(base, RPT)], tmp_v)

        def add(i, _):
            sl = pl.ds(i * 16, 16)
            col_v[sl] = col_v[sl] + tmp_v[sl]
            return 0
        lax.fori_loop(0, RPT // 16, add, 0)
        return 0
    lax.fori_loop(0, TPC, acc_t, 0)

    pltpu.sync_copy(col_v, deg_out.at[c, pl.ds(base, RPT)])


@functools.partial(
    pl.kernel,
    out_type=jax.ShapeDtypeStruct((NC, NP_, DH), _f32),
    mesh=_MESH,
    scratch_types=[
        pltpu.VMEM((NBMAX, 128), _i32),
        pltpu.VMEM((NBMAX, 128), _i32),
        pltpu.VMEM((128, DH), _f32),
        pltpu.VMEM((128, DH), _f32),
        pltpu.VMEM_SHARED((NP_, DH), _f32),
        pltpu.SemaphoreType.DMA,
        pltpu.SemaphoreType.DMA,
    ],
    compiler_params=_SC_PARAMS,
)
def _scatter_kernel(src2d, dst2d, table, accp, src_v, dst_v, rowa, rowb,
                    shared, gsa, gsb):
    c = lax.axis_index("c")
    s = lax.axis_index("s")
    nb = jnp.where(c == 0, NB0, NB1)
    row0 = c * TPC * NB0 + s * nb
    z16 = jnp.zeros((16,), _f32)
    base = s * RPT

    def zr(i, _):
        def zc(k, _):
            rowa[i, pl.ds(k * 16, 16)] = z16
            return 0
        lax.fori_loop(0, DH // 16, zc, 0)
        return 0
    lax.fori_loop(0, 128, zr, 0)

    def zs(q, _):
        pltpu.sync_copy(rowa, shared.at[pl.ds(base + q * 128, 128)])
        return 0
    lax.fori_loop(0, RPT // 128, zs, 0)
    plsc.subcore_barrier()

    pltpu.sync_copy(src2d.at[pl.ds(row0, NBMAX)], src_v)
    pltpu.sync_copy(dst2d.at[pl.ds(row0, NBMAX)], dst_v)

    def pro(t, _):
        pltpu.async_copy(table.at[src_v.at[t]], rowa, gsa)
        return 0
    lax.fori_loop(0, 1, pro, 0)

    def eb(jj, _):
        j = jj * 2
        pltpu.make_async_copy(table.at[src_v.at[j]], rowa, gsa).wait()
        pltpu.async_copy(table.at[src_v.at[j + 1]], rowb, gsb)
        pltpu.sync_copy(rowa, shared.at[dst_v.at[j]], add=True)
        pltpu.make_async_copy(table.at[src_v.at[j + 1]], rowb, gsb).wait()
        pltpu.async_copy(table.at[src_v.at[j + 2]], rowa, gsa)
        pltpu.sync_copy(rowb, shared.at[dst_v.at[j + 1]], add=True)
        return 0
    lax.fori_loop(0, nb // 2 - 1, eb, 0)

    def epi(t, _):
        j = nb - 2 + t
        pltpu.make_async_copy(table.at[src_v.at[j]], rowa, gsa).wait()
        pltpu.async_copy(table.at[src_v.at[j + 1]], rowb, gsb)
        pltpu.sync_copy(rowa, shared.at[dst_v.at[j]], add=True)
        pltpu.make_async_copy(table.at[src_v.at[j + 1]], rowb, gsb).wait()
        pltpu.sync_copy(rowb, shared.at[dst_v.at[j + 1]], add=True)
        return 0
    lax.fori_loop(0, 1, epi, 0)
    plsc.subcore_barrier()

    def co(q, _):
        pltpu.sync_copy(shared.at[pl.ds(base + q * 128, 128)], rowa)
        pltpu.sync_copy(rowa, accp.at[c, pl.ds(base + q * 128, 128)])
        return 0
    lax.fori_loop(0, RPT // 128, co, 0)


def _lnt(x, s, b, eps=1e-5):
    m = jnp.mean(x, axis=-1, keepdims=True)
    v = jnp.mean((x - m) ** 2, axis=-1, keepdims=True)
    return (x - m) * lax.rsqrt(v + eps) * s + b


def _tc_b(xp, w1, degp, hl_o, hr_o, dinv_o):
    deg = degp[:, 0:1] + degp[:, 1:2]
    ii = lax.broadcasted_iota(_i32, (NP_, 1), 0)
    deg = deg + jnp.where(ii < N, 1.0, 0.0) - jnp.where(ii == PAD, float(NPADE), 0.0)
    dinv = jnp.where(deg > 0, lax.rsqrt(deg), 0.0)
    h1 = dinv * jnp.dot(xp[...], w1[...], preferred_element_type=_f32)
    hl_o[...] = h1[:, :DH]
    hr_o[...] = h1[:, DH:]
    dinv_o[...] = dinv


_TC_PARAMS = pltpu.CompilerParams(vmem_limit_bytes=100 * 1024 * 1024)

_B_CALL = pl.pallas_call(
    _tc_b,
    out_shape=[jax.ShapeDtypeStruct((NP_, DH), _f32),
               jax.ShapeDtypeStruct((NP_, DH), _f32),
               jax.ShapeDtypeStruct((NP_, 1), _f32)],
    compiler_params=_TC_PARAMS,
)


def _tc_c(accl, accr, hl, hr, dinv, w2, b1, s1, bb1, h2l_o, h2r_o):
    a = jnp.concatenate([accl[0] + accl[1] + hl[...],
                         accr[0] + accr[1] + hr[...]], axis=1)
    out1 = dinv[...] * a + b1[...]
    h2 = _lnt(jnp.tanh(out1), s1[...], bb1[...])
    h2p = dinv[...] * jnp.dot(h2, w2[...], preferred_element_type=_f32)
    h2l_o[...] = h2p[:, :DH]
    h2r_o[...] = h2p[:, DH:]


_C_CALL = pl.pallas_call(
    _tc_c,
    out_shape=[jax.ShapeDtypeStruct((NP_, DH), _f32),
               jax.ShapeDtypeStruct((NP_, DH), _f32)],
    compiler_params=_TC_PARAMS,
)


def _tc_d(accl, accr, h2l, h2r, dinv, bcol, brow,
          b2, s2, bb2, sp, bp,
          gw1, gb1, gw2, gb2, gw3, gb3, gw4r, gb4,
          mw1, mb1, ms1, mbb1, mw2, mb2, ms2, mbb2, mw3, mb3,
          out_ref):
    a = jnp.concatenate([accl[0] + accl[1] + h2l[...],
                         accr[0] + accr[1] + h2r[...]], axis=1)
    out2 = dinv[...] * a + b2[...]
    h3 = _lnt(jnp.tanh(out2), s2[...], bb2[...])
    hp = _lnt(h3, sp[...], bp[...])
    g = jnp.tanh(jnp.dot(hp, gw1[...], preferred_element_type=_f32) + gb1[...])
    g = jnp.tanh(jnp.dot(g, gw2[...], preferred_element_type=_f32) + gb2[...])
    g = jnp.tanh(jnp.dot(g, gw3[...], preferred_element_type=_f32) + gb3[...])
    g = jnp.sum(g * gw4r[...], axis=1, keepdims=True) + gb4[0, 0]
    iota_row = lax.broadcasted_iota(_i32, (1, G), 1)
    maskT = (bcol[...] == iota_row).astype(_f32)
    gmax = jnp.max(jnp.where(maskT > 0, g, -jnp.inf), axis=0, keepdims=True)
    gmax = jnp.where(gmax > -1e30, gmax, 0.0)
    gmax_n = jnp.sum(maskT * gmax, axis=1, keepdims=True)
    e = jnp.exp(g - gmax_n)
    esum = jnp.sum(maskT * e, axis=0, keepdims=True)
    esum_n = jnp.sum(maskT * esum, axis=1, keepdims=True)
    alpha = e / (esum_n + 1e-16)
    iota_col = lax.broadcasted_iota(_i32, (G, 1), 0)
    mask2 = (brow[...] == iota_col).astype(_f32)
    pooled = jnp.dot(mask2, alpha * hp, preferred_element_type=_f32)
    m = jnp.dot(pooled, mw1[...], preferred_element_type=_f32) + mb1[...]
    m = jnp.tanh(_lnt(m, ms1[...], mbb1[...]))
    m = jnp.dot(m, mw2[...], preferred_element_type=_f32) + mb2[...]
    m = jnp.tanh(_lnt(m, ms2[...], mbb2[...]))
    out_ref[...] = jnp.dot(m, mw3[...], preferred_element_type=_f32) + mb3[...]


_D_CALL = pl.pallas_call(
    _tc_d,
    out_shape=jax.ShapeDtypeStruct((G, G), _f32),
    compiler_params=_TC_PARAMS,
)


def _row(v):
    return v.reshape(1, -1)


def kernel(x, edge_index, batch, params):
    p = params
    src = edge_index[0].astype(_i32)
    dst = edge_index[1].astype(_i32)
    padi = jnp.full((NPADE + XROWS * 128,), PAD, _i32)
    src2d = jnp.concatenate([src, padi]).reshape(EROWS + XROWS, 128)
    dst2d = jnp.concatenate([dst, padi]).reshape(EROWS + XROWS, 128)
    xp = jnp.zeros((NP_, D), _f32).at[:N].set(x)
    bpad = jnp.concatenate([batch.astype(_i32), jnp.full((NP_ - N,), -1, _i32)])
    bcol = bpad.reshape(NP_, 1)
    brow = bpad.reshape(1, NP_)

    degp = _hist_kernel(dst2d)
    h1l, h1r, dinv = _B_CALL(xp, p['W1'], degp.T)
    a1l = _scatter_kernel(src2d, dst2d, h1l)
    a1r = _scatter_kernel(src2d, dst2d, h1r)
    h2l, h2r = _C_CALL(a1l, a1r, h1l, h1r, dinv, p['W2'],
                       _row(p['b1']), _row(p['ln1_s']), _row(p['ln1_b']))
    a2l = _scatter_kernel(src2d, dst2d, h2l)
    a2r = _scatter_kernel(src2d, dst2d, h2r)
    out = _D_CALL(a2l, a2r, h2l, h2r, dinv, bcol, brow,
                  _row(p['b2']), _row(p['ln2_s']), _row(p['ln2_b']),
                  _row(p['pn_s']), _row(p['pn_b']),
                  p['Gw1'], _row(p['Gb1']), p['Gw2'], _row(p['Gb2']),
                  p['Gw3'], _row(p['Gb3']), _row(p['Gw4'][:, 0]), p['Gb4'].reshape(1, 1),
                  p['Mw1'], _row(p['Mb1']), _row(p['Mln1_s']), _row(p['Mln1_b']),
                  p['Mw2'], _row(p['Mb2']), _row(p['Mln2_s']), _row(p['Mln2_b']),
                  p['Mw3'], _row(p['Mb3']))
    return out
